# Initial kernel scaffold; baseline (speedup 1.0000x reference)
import jax
import jax.numpy as jnp
from jax.experimental import pallas as pl


def kernel(x_seq, x_stats, edge_attr, conv_w, conv_b, seq_w, seq_b, stat_w, stat_b, fuse_w, fuse_b, type_emb, ef_w, ef_b, g0_wl, g0_wr, g0_we, g0_att, g0_b, g1_wl, g1_wr, g1_we, g1_att, g1_b, gate_w, gate_b, c1_w, c1_b, c2_w, c2_b, edge_index, batch):
    raise NotImplementedError("write your pallas kernel here")


# speedup vs baseline: 15.6975x; 15.6975x over previous
import functools

import jax
import jax.numpy as jnp
import numpy as np
from jax import lax
from jax.experimental import pallas as pl
from jax.experimental.pallas import tpu as pltpu
from jax.experimental.pallas import tpu_sc as plsc

N = 10000
E = 160000
L = 100
SD = 5
NK = 64
ED = 128
HD = 256
HEADS = 4
C = 64
NCLS = 10
NG = 64

NP = 10240
NWORK = 32
ECHUNK = 200
NB = 200
EB = 2000


def _head_onehot(rows, cols, transposed=False):
    if transposed:
        r = lax.broadcasted_iota(jnp.int32, (rows, cols), 0)
        c = lax.broadcasted_iota(jnp.int32, (rows, cols), 1) // C
    else:
        r = lax.broadcasted_iota(jnp.int32, (rows, cols), 0) // C
        c = lax.broadcasted_iota(jnp.int32, (rows, cols), 1)
    return (r == c).astype(jnp.float32)


def _node_body(xs_ref, st_ref, cw_ref, cb_ref, sw_ref, sb_ref, tw_ref, tb_ref,
               fw1_ref, fw2_ref, fb_ref, wl_ref, wr_ref,
               x_out, xl_out, xr_out):
    xs = xs_ref[...]
    cw = cw_ref[...]
    acc = jnp.zeros((NB, L - 4, NK), jnp.float32)
    for j in range(5):
        acc = acc + xs[:, j:j + (L - 4)][:, :, None] * cw[j][None, None, :]
    h = jnp.max(jax.nn.relu(acc + cb_ref[...][None, :, :]), axis=1)
    hs = jax.nn.relu(h @ sw_ref[...] + sb_ref[...])
    st = jax.nn.relu(st_ref[...] @ tw_ref[...] + tb_ref[...])
    x = hs @ fw1_ref[...] + st @ fw2_ref[...] + fb_ref[...]
    x_out[...] = x
    xl_out[...] = x @ wl_ref[...]
    xr_out[...] = x @ wr_ref[...]


def _node_stage(x_seq, x_stats, cw, cb, sw, sb, tw, tb, fw1, fw2, fb, wl, wr):
    grid = N // NB
    full = lambda a: pl.BlockSpec(a.shape, lambda i: (0,) * a.ndim)
    return pl.pallas_call(
        _node_body,
        grid=(grid,),
        in_specs=[
            pl.BlockSpec((NB, L), lambda i: (i, 0)),
            pl.BlockSpec((NB, SD), lambda i: (i, 0)),
            full(cw), full(cb), full(sw), full(sb), full(tw), full(tb),
            full(fw1), full(fw2), full(fb), full(wl), full(wr),
        ],
        out_specs=[pl.BlockSpec((NB, HD), lambda i: (i, 0))] * 3,
        out_shape=[jax.ShapeDtypeStruct((N, HD), jnp.float32)] * 3,
    )(x_seq, x_stats, cw, cb, sw, sb, tw, tb, fw1, fw2, fb, wl, wr)


def _edge_body(ea_ref, tep_ref, efw_ref, efb_ref, w0_ref, w1_ref,
               we0_out, we1_out):
    ea = ea_ref[...]
    et = ea[:, 0:1]
    dt = ea[:, 1:2]
    sel = (et == lax.broadcasted_iota(jnp.int32, (1, 3), 1).astype(jnp.float32))
    sel = sel.astype(jnp.float32)
    i2 = lax.broadcasted_iota(jnp.int32, (1, ED), 1).astype(jnp.float32)
    div = jnp.exp(i2 * (-2.0 * np.log(10000.0) / HD))
    pe = dt * div
    q = jnp.concatenate([jnp.sin(pe), jnp.cos(pe)], axis=1)
    ee = (sel @ tep_ref[...] + q) @ efw_ref[...] + efb_ref[...]
    we0_out[...] = ee @ w0_ref[...]
    we1_out[...] = ee @ w1_ref[...]


def _edge_stage(edge_attr, tep, efwp, efb, w0, w1):
    grid = E // EB
    full = lambda a: pl.BlockSpec(a.shape, lambda i: (0,) * a.ndim)
    return pl.pallas_call(
        _edge_body,
        grid=(grid,),
        in_specs=[
            pl.BlockSpec((EB, 2), lambda i: (i, 0)),
            full(tep), full(efwp), full(efb), full(w0), full(w1),
        ],
        out_specs=[pl.BlockSpec((EB, HD), lambda i: (i, 0))] * 2,
        out_shape=[jax.ShapeDtypeStruct((E, HD), jnp.float32)] * 2,
    )(edge_attr, tep, efwp, efb, w0, w1)


def _msg_body(xls_ref, xrd_ref, we_ref, att_ref, u0_out, u1_out, el_out):
    xls = xls_ref[...]
    m = xls + xrd_ref[...] + we_ref[...]
    m = jnp.where(m > 0, m, 0.2 * m)
    ma = m * att_ref[...]
    hsel = _head_onehot(HD, HEADS)
    logits = ma @ hsel
    el = jnp.exp(logits)
    eb = el @ _head_onehot(HEADS, HD, transposed=True)
    u = xls * eb
    u0_out[...] = u[:, :ED]
    u1_out[...] = u[:, ED:]
    p16 = (lax.broadcasted_iota(jnp.int32, (HEADS, 16), 0) ==
           lax.broadcasted_iota(jnp.int32, (HEADS, 16), 1)).astype(jnp.float32)
    el_out[...] = el @ p16


def _msg_stage(xls, xrd, we, att_flat):
    grid = E // EB
    full = lambda a: pl.BlockSpec(a.shape, lambda i: (0,) * a.ndim)
    return pl.pallas_call(
        _msg_body,
        grid=(grid,),
        in_specs=[
            pl.BlockSpec((EB, HD), lambda i: (i, 0)),
            pl.BlockSpec((EB, HD), lambda i: (i, 0)),
            pl.BlockSpec((EB, HD), lambda i: (i, 0)),
            full(att_flat),
        ],
        out_specs=[
            pl.BlockSpec((EB, ED), lambda i: (i, 0)),
            pl.BlockSpec((EB, ED), lambda i: (i, 0)),
            pl.BlockSpec((EB, 16), lambda i: (i, 0)),
        ],
        out_shape=[
            jax.ShapeDtypeStruct((E, ED), jnp.float32),
            jax.ShapeDtypeStruct((E, ED), jnp.float32),
            jax.ShapeDtypeStruct((E, 16), jnp.float32),
        ],
    )(xls, xrd, we, att_flat)


def _norm_body(p0_ref, p1_ref, pe_ref, b_ref, *rest):
    if len(rest) == 5:
        wl_ref, wr_ref, x_out, xl_out, xr_out = rest
    else:
        (x_out,) = rest
        wl_ref = wr_ref = None
    p0 = p0_ref[...]
    p1 = p1_ref[...]
    pe = pe_ref[...]
    u = jnp.concatenate([p0[0] + p0[1], p1[0] + p1[1]], axis=1)
    el = (pe[0] + pe[1])[:, :HEADS]
    den = el @ _head_onehot(HEADS, HD, transposed=True) + 1e-16
    out = u / den + b_ref[...]
    x = jnp.where(out > 0, out, jnp.exp(out) - 1.0)
    x_out[...] = x
    if wl_ref is not None:
        xl_out[...] = x @ wl_ref[...]
        xr_out[...] = x @ wr_ref[...]


def _norm_stage(p0, p1, pe, b, wl=None, wr=None):
    grid = NP // 256
    full = lambda a: pl.BlockSpec(a.shape, lambda i: (0,) * a.ndim)
    n_out = 3 if wl is not None else 1
    in_arrays = [p0, p1, pe, b] + ([wl, wr] if wl is not None else [])
    in_specs = [
        pl.BlockSpec((2, 256, ED), lambda i: (0, i, 0)),
        pl.BlockSpec((2, 256, ED), lambda i: (0, i, 0)),
        pl.BlockSpec((2, 256, 16), lambda i: (0, i, 0)),
        full(b),
    ] + ([full(wl), full(wr)] if wl is not None else [])
    return pl.pallas_call(
        _norm_body,
        grid=(grid,),
        in_specs=in_specs,
        out_specs=[pl.BlockSpec((256, HD), lambda i: (i, 0))] * n_out,
        out_shape=[jax.ShapeDtypeStruct((NP, HD), jnp.float32)] * n_out,
    )(*in_arrays)


def _pool_body(x_ref, bat_ref, gw_ref, gb_ref, c1w_ref, c1b_ref,
               c2w_ref, c2b_ref, out_ref):
    x = x_ref[...]
    gate = jax.nn.sigmoid(x @ gw_ref[...] + gb_ref[...])
    bat = bat_ref[...]
    oh = (bat == lax.broadcasted_iota(jnp.int32, (1, NG), 1)).astype(jnp.float32)
    eg = jnp.exp(gate)
    dn = (((0,), (0,)), ((), ()))
    s = lax.dot_general(oh, eg, dn)
    den = oh @ s + 1e-16
    ga = eg / den
    g = lax.dot_general(oh, ga * x, dn)
    h1 = g @ c1w_ref[...] + c1b_ref[...]
    h1 = jnp.where(h1 > 0, h1, 0.01 * h1)
    out_ref[...] = h1 @ c2w_ref[...] + c2b_ref[...]


def _pool_stage(x, bat, gw, gb, c1w, c1b, c2w, c2b):
    full = lambda a: pl.BlockSpec(a.shape, lambda i: (0,) * a.ndim)
    return pl.pallas_call(
        _pool_body,
        grid=(1,),
        in_specs=[full(a) for a in (x, bat, gw, gb, c1w, c1b, c2w, c2b)],
        out_specs=full(jnp.zeros((NG, NCLS))),
        out_shape=jax.ShapeDtypeStruct((NG, NCLS), jnp.float32),
    )(x, bat, gw, gb, c1w, c1b, c2w, c2b)


def _make_gather(ncols):
    mesh = plsc.VectorSubcoreMesh(core_axis_name="c", subcore_axis_name="s")
    per_w = E // NWORK
    n_it = per_w // ECHUNK

    @functools.partial(
        pl.kernel, mesh=mesh,
        out_type=jax.ShapeDtypeStruct((E, ncols), jnp.float32),
        scratch_types=[
            pltpu.VMEM((ECHUNK,), jnp.int32),
            pltpu.VMEM((ECHUNK, ncols), jnp.float32),
            pltpu.SemaphoreType.DMA,
        ],
    )
    def gk(table_hbm, idx_hbm, out_hbm, idx_v, rows_v, sem):
        wid = lax.axis_index("s") * 2 + lax.axis_index("c")
        base = wid * per_w

        def body(i, carry):
            b = base + i * ECHUNK
            pltpu.sync_copy(idx_hbm.at[pl.ds(b, ECHUNK)], idx_v)
            pltpu.async_copy(table_hbm.at[idx_v], rows_v, sem).wait()
            pltpu.sync_copy(rows_v, out_hbm.at[pl.ds(b, ECHUNK)])
            return carry

        lax.fori_loop(0, n_it, body, 0)

    return gk


def _make_scatter(ncols):
    mesh = plsc.VectorSubcoreMesh(core_axis_name="c", subcore_axis_name="s")
    per_w = E // NWORK
    n_it = per_w // ECHUNK
    zrows = NP // 16

    @functools.partial(
        pl.kernel, mesh=mesh,
        out_type=jax.ShapeDtypeStruct((2 * NP, ncols), jnp.float32),
        scratch_types=[
            pltpu.VMEM((ECHUNK,), jnp.int32),
            pltpu.VMEM((ECHUNK, ncols), jnp.float32),
            pltpu.VMEM_SHARED((NP, ncols), jnp.float32),
        ],
    )
    def sk(rows_hbm, idx_hbm, zeros_hbm, out_hbm, idx_v, rows_v, acc_sh):
        cid = lax.axis_index("c")
        sid = lax.axis_index("s")
        wid = sid * 2 + cid
        base = wid * per_w
        pltpu.sync_copy(zeros_hbm.at[pl.ds(sid * zrows, zrows)],
                        acc_sh.at[pl.ds(sid * zrows, zrows)])
        plsc.subcore_barrier()

        def body(i, carry):
            b = base + i * ECHUNK
            pltpu.sync_copy(rows_hbm.at[pl.ds(b, ECHUNK)], rows_v)
            pltpu.sync_copy(idx_hbm.at[pl.ds(b, ECHUNK)], idx_v)
            pltpu.sync_copy(rows_v, acc_sh.at[idx_v], add=True)
            return carry

        lax.fori_loop(0, n_it, body, 0)
        plsc.subcore_barrier()
        pltpu.sync_copy(acc_sh.at[pl.ds(sid * zrows, zrows)],
                        out_hbm.at[pl.ds(cid * NP + sid * zrows, zrows)])

    return sk


def kernel(x_seq, x_stats, edge_attr, conv_w, conv_b, seq_w, seq_b, stat_w,
           stat_b, fuse_w, fuse_b, type_emb, ef_w, ef_b,
           g0_wl, g0_wr, g0_we, g0_att, g0_b,
           g1_wl, g1_wr, g1_we, g1_att, g1_b,
           gate_w, gate_b, c1_w, c1_b, c2_w, c2_b, edge_index, batch):
    f32 = jnp.float32
    src = edge_index[0].astype(jnp.int32)
    dst = edge_index[1].astype(jnp.int32)

    perm = np.concatenate([np.arange(0, HD, 2), np.arange(1, HD, 2)])
    tep = type_emb[:, perm]
    efwp = ef_w[perm, :]
    cw = conv_w[:, 0, :].T
    row = lambda v: v.reshape(1, -1).astype(f32)

    x, xl, xr = _node_stage(
        x_seq, x_stats, cw, row(conv_b), seq_w, row(seq_b), stat_w,
        row(stat_b), fuse_w[:ED], fuse_w[ED:], row(fuse_b), g0_wl, g0_wr)

    we0, we1 = _edge_stage(edge_attr, tep, efwp, row(ef_b), g0_we, g1_we)

    gather = _make_gather(HD)
    scat_u = _make_scatter(ED)
    scat_e = _make_scatter(16)
    zeros_u = jnp.zeros((NP, ED), f32)
    zeros_e = jnp.zeros((NP, 16), f32)

    for (we, att, b, wl_next, wr_next) in (
            (we0, g0_att, g0_b, g1_wl, g1_wr),
            (we1, g1_att, g1_b, None, None)):
        xls = gather(xl, src)
        xrd = gather(xr, dst)
        u0, u1, el16 = _msg_stage(xls, xrd, we, att.reshape(1, HD))
        p0 = scat_u(u0, dst, zeros_u).reshape(2, NP, ED)
        p1 = scat_u(u1, dst, zeros_u).reshape(2, NP, ED)
        pe = scat_e(el16, dst, zeros_e).reshape(2, NP, 16)
        if wl_next is not None:
            xnew, xl, xr = _norm_stage(p0, p1, pe, row(b), wl_next, wr_next)
        else:
            (xnew,) = _norm_stage(p0, p1, pe, row(b))

    bat = batch.astype(jnp.int32).reshape(N, 1)
    return _pool_stage(xnew[:N], bat, gate_w, row(gate_b),
                       c1_w, row(c1_b), c2_w, row(c2_b))


# speedup vs baseline: 15.8182x; 1.0077x over previous
import functools

import jax
import jax.numpy as jnp
import numpy as np
from jax import lax
from jax.experimental import pallas as pl
from jax.experimental.pallas import tpu as pltpu
from jax.experimental.pallas import tpu_sc as plsc

N = 10000
E = 160000
L = 100
SD = 5
NK = 64
ED = 128
HD = 256
HEADS = 4
C = 64
NCLS = 10
NG = 64

NP = 10240
NWORK = 32
ECHUNK = 200
NB = 200
EB = 2000


def _head_onehot(rows, cols, transposed=False):
    if transposed:
        r = lax.broadcasted_iota(jnp.int32, (rows, cols), 0)
        c = lax.broadcasted_iota(jnp.int32, (rows, cols), 1) // C
    else:
        r = lax.broadcasted_iota(jnp.int32, (rows, cols), 0) // C
        c = lax.broadcasted_iota(jnp.int32, (rows, cols), 1)
    return (r == c).astype(jnp.float32)


def _node_body(xs_ref, st_ref, cw_ref, cb_ref, sw_ref, sb_ref, tw_ref, tb_ref,
               fw1_ref, fw2_ref, fb_ref, wl_ref, wr_ref,
               x_out, xl_out, xr_out):
    xs = xs_ref[...]
    cw = cw_ref[...]
    acc = jnp.zeros((NB, L - 4, NK), jnp.float32)
    for j in range(5):
        acc = acc + xs[:, j:j + (L - 4)][:, :, None] * cw[j][None, None, :]
    h = jnp.max(jax.nn.relu(acc + cb_ref[...][None, :, :]), axis=1)
    hs = jax.nn.relu(h @ sw_ref[...] + sb_ref[...])
    st = jax.nn.relu(st_ref[...] @ tw_ref[...] + tb_ref[...])
    x = hs @ fw1_ref[...] + st @ fw2_ref[...] + fb_ref[...]
    x_out[...] = x
    xl_out[...] = x @ wl_ref[...]
    xr_out[...] = x @ wr_ref[...]


def _node_stage(x_seq, x_stats, cw, cb, sw, sb, tw, tb, fw1, fw2, fb, wl, wr):
    grid = N // NB
    full = lambda a: pl.BlockSpec(a.shape, lambda i: (0,) * a.ndim)
    return pl.pallas_call(
        _node_body,
        grid=(grid,),
        in_specs=[
            pl.BlockSpec((NB, L), lambda i: (i, 0)),
            pl.BlockSpec((NB, SD), lambda i: (i, 0)),
            full(cw), full(cb), full(sw), full(sb), full(tw), full(tb),
            full(fw1), full(fw2), full(fb), full(wl), full(wr),
        ],
        out_specs=[pl.BlockSpec((NB, HD), lambda i: (i, 0))] * 3,
        out_shape=[jax.ShapeDtypeStruct((N, HD), jnp.float32)] * 3,
    )(x_seq, x_stats, cw, cb, sw, sb, tw, tb, fw1, fw2, fb, wl, wr)


def _edge_body(ea_ref, tep_ref, efw_ref, efb_ref, w0_ref, w1_ref,
               we0_out, we1_out):
    ea = ea_ref[...]
    et = ea[:, 0:1]
    dt = ea[:, 1:2]
    sel = (et == lax.broadcasted_iota(jnp.int32, (1, 3), 1).astype(jnp.float32))
    sel = sel.astype(jnp.float32)
    i2 = lax.broadcasted_iota(jnp.int32, (1, ED), 1).astype(jnp.float32)
    div = jnp.exp(i2 * (-2.0 * np.log(10000.0) / HD))
    pe = dt * div
    q = jnp.concatenate([jnp.sin(pe), jnp.cos(pe)], axis=1)
    ee = (sel @ tep_ref[...] + q) @ efw_ref[...] + efb_ref[...]
    we0_out[...] = ee @ w0_ref[...]
    we1_out[...] = ee @ w1_ref[...]


def _edge_stage(edge_attr, tep, efwp, efb, w0, w1):
    grid = E // EB
    full = lambda a: pl.BlockSpec(a.shape, lambda i: (0,) * a.ndim)
    return pl.pallas_call(
        _edge_body,
        grid=(grid,),
        in_specs=[
            pl.BlockSpec((EB, 2), lambda i: (i, 0)),
            full(tep), full(efwp), full(efb), full(w0), full(w1),
        ],
        out_specs=[pl.BlockSpec((EB, HD), lambda i: (i, 0))] * 2,
        out_shape=[jax.ShapeDtypeStruct((E, HD), jnp.float32)] * 2,
    )(edge_attr, tep, efwp, efb, w0, w1)


def _msg_body(xls_ref, xrd_ref, we_ref, att_ref, u0_out, u1_out, el_out):
    xls = xls_ref[...]
    m = xls + xrd_ref[...] + we_ref[...]
    m = jnp.where(m > 0, m, 0.2 * m)
    ma = m * att_ref[...]
    hsel = _head_onehot(HD, HEADS)
    logits = ma @ hsel
    el = jnp.exp(logits)
    eb = el @ _head_onehot(HEADS, HD, transposed=True)
    u = xls * eb
    u0_out[...] = u[:, :ED]
    u1_out[...] = u[:, ED:]
    p16 = (lax.broadcasted_iota(jnp.int32, (HEADS, 16), 0) ==
           lax.broadcasted_iota(jnp.int32, (HEADS, 16), 1)).astype(jnp.float32)
    el_out[...] = el @ p16


def _msg_stage(xls, xrd, we, att_flat):
    grid = E // EB
    full = lambda a: pl.BlockSpec(a.shape, lambda i: (0,) * a.ndim)
    return pl.pallas_call(
        _msg_body,
        grid=(grid,),
        in_specs=[
            pl.BlockSpec((EB, HD), lambda i: (i, 0)),
            pl.BlockSpec((EB, HD), lambda i: (i, 0)),
            pl.BlockSpec((EB, HD), lambda i: (i, 0)),
            full(att_flat),
        ],
        out_specs=[
            pl.BlockSpec((EB, ED), lambda i: (i, 0)),
            pl.BlockSpec((EB, ED), lambda i: (i, 0)),
            pl.BlockSpec((EB, 16), lambda i: (i, 0)),
        ],
        out_shape=[
            jax.ShapeDtypeStruct((E, ED), jnp.float32),
            jax.ShapeDtypeStruct((E, ED), jnp.float32),
            jax.ShapeDtypeStruct((E, 16), jnp.float32),
        ],
    )(xls, xrd, we, att_flat)


def _norm_body(p0_ref, p1_ref, pe_ref, b_ref, *rest):
    if len(rest) == 5:
        wl_ref, wr_ref, x_out, xl_out, xr_out = rest
    else:
        (x_out,) = rest
        wl_ref = wr_ref = None
    p0 = p0_ref[...]
    p1 = p1_ref[...]
    pe = pe_ref[...]
    u = jnp.concatenate([p0[0] + p0[1], p1[0] + p1[1]], axis=1)
    el = (pe[0] + pe[1])[:, :HEADS]
    den = el @ _head_onehot(HEADS, HD, transposed=True) + 1e-16
    out = u / den + b_ref[...]
    x = jnp.where(out > 0, out, jnp.exp(out) - 1.0)
    x_out[...] = x
    if wl_ref is not None:
        xl_out[...] = x @ wl_ref[...]
        xr_out[...] = x @ wr_ref[...]


def _norm_stage(p0, p1, pe, b, wl=None, wr=None):
    grid = NP // 256
    full = lambda a: pl.BlockSpec(a.shape, lambda i: (0,) * a.ndim)
    n_out = 3 if wl is not None else 1
    in_arrays = [p0, p1, pe, b] + ([wl, wr] if wl is not None else [])
    in_specs = [
        pl.BlockSpec((2, 256, ED), lambda i: (0, i, 0)),
        pl.BlockSpec((2, 256, ED), lambda i: (0, i, 0)),
        pl.BlockSpec((2, 256, 16), lambda i: (0, i, 0)),
        full(b),
    ] + ([full(wl), full(wr)] if wl is not None else [])
    return pl.pallas_call(
        _norm_body,
        grid=(grid,),
        in_specs=in_specs,
        out_specs=[pl.BlockSpec((256, HD), lambda i: (i, 0))] * n_out,
        out_shape=[jax.ShapeDtypeStruct((NP, HD), jnp.float32)] * n_out,
    )(*in_arrays)


def _pool_body(x_ref, bat_ref, gw_ref, gb_ref, c1w_ref, c1b_ref,
               c2w_ref, c2b_ref, out_ref):
    x = x_ref[...]
    gate = jax.nn.sigmoid(x @ gw_ref[...] + gb_ref[...])
    bat = bat_ref[...]
    oh = (bat == lax.broadcasted_iota(jnp.int32, (1, NG), 1)).astype(jnp.float32)
    eg = jnp.exp(gate)
    dn = (((0,), (0,)), ((), ()))
    s = lax.dot_general(oh, eg, dn)
    den = oh @ s + 1e-16
    ga = eg / den
    g = lax.dot_general(oh, ga * x, dn)
    h1 = g @ c1w_ref[...] + c1b_ref[...]
    h1 = jnp.where(h1 > 0, h1, 0.01 * h1)
    out_ref[...] = h1 @ c2w_ref[...] + c2b_ref[...]


def _pool_stage(x, bat, gw, gb, c1w, c1b, c2w, c2b):
    full = lambda a: pl.BlockSpec(a.shape, lambda i: (0,) * a.ndim)
    return pl.pallas_call(
        _pool_body,
        grid=(1,),
        in_specs=[full(a) for a in (x, bat, gw, gb, c1w, c1b, c2w, c2b)],
        out_specs=full(jnp.zeros((NG, NCLS))),
        out_shape=jax.ShapeDtypeStruct((NG, NCLS), jnp.float32),
    )(x, bat, gw, gb, c1w, c1b, c2w, c2b)


def _make_gather2(ncols):
    mesh = plsc.VectorSubcoreMesh(core_axis_name="c", subcore_axis_name="s")
    per_w = E // NWORK
    n_it = per_w // ECHUNK

    @functools.partial(
        pl.kernel, mesh=mesh,
        out_type=[jax.ShapeDtypeStruct((E, ncols), jnp.float32)] * 2,
        scratch_types=[
            pltpu.VMEM((ECHUNK,), jnp.int32),
            pltpu.VMEM((ECHUNK,), jnp.int32),
            pltpu.VMEM((ECHUNK, ncols), jnp.float32),
            pltpu.VMEM((ECHUNK, ncols), jnp.float32),
            pltpu.SemaphoreType.DMA,
            pltpu.SemaphoreType.DMA,
        ],
    )
    def gk(tl_hbm, tr_hbm, src_hbm, dst_hbm, outl_hbm, outr_hbm,
           si_v, di_v, lrows_v, rrows_v, sem0, sem1):
        wid = lax.axis_index("s") * 2 + lax.axis_index("c")
        base = wid * per_w

        def body(i, carry):
            b = base + i * ECHUNK
            pltpu.sync_copy(src_hbm.at[pl.ds(b, ECHUNK)], si_v)
            pltpu.sync_copy(dst_hbm.at[pl.ds(b, ECHUNK)], di_v)
            cl = pltpu.async_copy(tl_hbm.at[si_v], lrows_v, sem0)
            cr = pltpu.async_copy(tr_hbm.at[di_v], rrows_v, sem1)
            cl.wait()
            pltpu.sync_copy(lrows_v, outl_hbm.at[pl.ds(b, ECHUNK)])
            cr.wait()
            pltpu.sync_copy(rrows_v, outr_hbm.at[pl.ds(b, ECHUNK)])
            return carry

        lax.fori_loop(0, n_it, body, 0)

    return gk


def _make_scatter(ncols):
    mesh = plsc.VectorSubcoreMesh(core_axis_name="c", subcore_axis_name="s")
    per_w = E // NWORK
    n_it = per_w // ECHUNK
    zrows = NP // 16

    @functools.partial(
        pl.kernel, mesh=mesh,
        out_type=jax.ShapeDtypeStruct((2 * NP, ncols), jnp.float32),
        scratch_types=[
            pltpu.VMEM((ECHUNK,), jnp.int32),
            pltpu.VMEM((ECHUNK, ncols), jnp.float32),
            pltpu.VMEM_SHARED((NP, ncols), jnp.float32),
        ],
    )
    def sk(rows_hbm, idx_hbm, zeros_hbm, out_hbm, idx_v, rows_v, acc_sh):
        cid = lax.axis_index("c")
        sid = lax.axis_index("s")
        wid = sid * 2 + cid
        base = wid * per_w
        pltpu.sync_copy(zeros_hbm.at[pl.ds(sid * zrows, zrows)],
                        acc_sh.at[pl.ds(sid * zrows, zrows)])
        plsc.subcore_barrier()

        def body(i, carry):
            b = base + i * ECHUNK
            pltpu.sync_copy(rows_hbm.at[pl.ds(b, ECHUNK)], rows_v)
            pltpu.sync_copy(idx_hbm.at[pl.ds(b, ECHUNK)], idx_v)
            pltpu.sync_copy(rows_v, acc_sh.at[idx_v], add=True)
            return carry

        lax.fori_loop(0, n_it, body, 0)
        plsc.subcore_barrier()
        pltpu.sync_copy(acc_sh.at[pl.ds(sid * zrows, zrows)],
                        out_hbm.at[pl.ds(cid * NP + sid * zrows, zrows)])

    return sk


def kernel(x_seq, x_stats, edge_attr, conv_w, conv_b, seq_w, seq_b, stat_w,
           stat_b, fuse_w, fuse_b, type_emb, ef_w, ef_b,
           g0_wl, g0_wr, g0_we, g0_att, g0_b,
           g1_wl, g1_wr, g1_we, g1_att, g1_b,
           gate_w, gate_b, c1_w, c1_b, c2_w, c2_b, edge_index, batch):
    f32 = jnp.float32
    src = edge_index[0].astype(jnp.int32)
    dst = edge_index[1].astype(jnp.int32)

    perm = np.concatenate([np.arange(0, HD, 2), np.arange(1, HD, 2)])
    tep = type_emb[:, perm]
    efwp = ef_w[perm, :]
    cw = conv_w[:, 0, :].T
    row = lambda v: v.reshape(1, -1).astype(f32)

    x, xl, xr = _node_stage(
        x_seq, x_stats, cw, row(conv_b), seq_w, row(seq_b), stat_w,
        row(stat_b), fuse_w[:ED], fuse_w[ED:], row(fuse_b), g0_wl, g0_wr)

    we0, we1 = _edge_stage(edge_attr, tep, efwp, row(ef_b), g0_we, g1_we)

    gather2 = _make_gather2(HD)
    scat_u = _make_scatter(ED)
    scat_e = _make_scatter(16)
    zeros_u = jnp.zeros((NP, ED), f32)
    zeros_e = jnp.zeros((NP, 16), f32)

    for (we, att, b, wl_next, wr_next) in (
            (we0, g0_att, g0_b, g1_wl, g1_wr),
            (we1, g1_att, g1_b, None, None)):
        xls, xrd = gather2(xl, xr, src, dst)
        u0, u1, el16 = _msg_stage(xls, xrd, we, att.reshape(1, HD))
        p0 = scat_u(u0, dst, zeros_u).reshape(2, NP, ED)
        p1 = scat_u(u1, dst, zeros_u).reshape(2, NP, ED)
        pe = scat_e(el16, dst, zeros_e).reshape(2, NP, 16)
        if wl_next is not None:
            xnew, xl, xr = _norm_stage(p0, p1, pe, row(b), wl_next, wr_next)
        else:
            (xnew,) = _norm_stage(p0, p1, pe, row(b))

    bat = batch.astype(jnp.int32).reshape(N, 1)
    return _pool_stage(xnew[:N], bat, gate_w, row(gate_b),
                       c1_w, row(c1_b), c2_w, row(c2_b))
